# baseline (speedup 1.0000x reference)
import jax
import jax.numpy as jnp
from jax.experimental import pallas as pl
from jax.experimental.pallas import tpu as pltpu

_N_GROUPS = 8192
_DIM = 256
_BETA = 0.99
_TOKENS = 65536
_TM = 256


def _gfn_body(gf_ref, gfn_ref):
    gf = gf_ref[...]
    ns = jnp.sum(gf * gf, axis=1, keepdims=True)
    rnorm = 1.0 / jnp.maximum(jnp.sqrt(ns), 1e-12)
    gfn_ref[...] = (gf * rnorm).astype(jnp.bfloat16)


def _accum_prev(sums_ref, counts_ref, oh, xp):
    sums_ref[...] += jax.lax.dot_general(
        oh, xp, (((0,), (0,)), ((), ())),
        preferred_element_type=jnp.float32)
    counts_ref[...] += jnp.sum(oh.astype(jnp.float32), axis=0,
                               keepdims=True)


def _assign_accum_body(x_ref, gfn_ref, sums_ref, counts_ref, oh_ref, xp_ref):
    i = pl.program_id(0)
    n = pl.num_programs(0)

    @pl.when(i == 0)
    def _init():
        sums_ref[...] = jnp.zeros_like(sums_ref)
        counts_ref[...] = jnp.zeros_like(counts_ref)

    @pl.when(i > 0)
    def _drain():
        _accum_prev(sums_ref, counts_ref, oh_ref[...], xp_ref[...])

    x = x_ref[...].astype(jnp.bfloat16)
    logits = jax.lax.dot_general(
        x, gfn_ref[...], (((1,), (1,)), ((), ())),
        preferred_element_type=jnp.float32)
    rowmax = jnp.max(logits, axis=1, keepdims=True)
    oh = (logits == rowmax).astype(jnp.bfloat16)
    oh_ref[...] = oh
    xp_ref[...] = x

    @pl.when(i == n - 1)
    def _tail():
        _accum_prev(sums_ref, counts_ref, oh, x)


def _blend_body(gf_ref, sums_ref, cnt_ref, out_ref):
    r = 1.0 / jnp.maximum(cnt_ref[...], 1.0)
    out_ref[...] = _BETA * gf_ref[...] + (1.0 - _BETA) * sums_ref[...] * r


@jax.jit
def kernel(x, group_features):
    gfn = pl.pallas_call(
        _gfn_body,
        in_specs=[pl.BlockSpec((_N_GROUPS, _DIM), lambda: (0, 0))],
        out_specs=pl.BlockSpec((_N_GROUPS, _DIM), lambda: (0, 0)),
        out_shape=jax.ShapeDtypeStruct((_N_GROUPS, _DIM), jnp.bfloat16),
    )(group_features)

    grid = _TOKENS // _TM
    sums, counts = pl.pallas_call(
        _assign_accum_body,
        grid=(grid,),
        in_specs=[
            pl.BlockSpec((_TM, _DIM), lambda i: (i, 0)),
            pl.BlockSpec((_N_GROUPS, _DIM), lambda i: (0, 0)),
        ],
        out_specs=[
            pl.BlockSpec((_N_GROUPS, _DIM), lambda i: (0, 0)),
            pl.BlockSpec((1, _N_GROUPS), lambda i: (0, 0)),
        ],
        out_shape=[
            jax.ShapeDtypeStruct((_N_GROUPS, _DIM), jnp.float32),
            jax.ShapeDtypeStruct((1, _N_GROUPS), jnp.float32),
        ],
        scratch_shapes=[
            pltpu.VMEM((_TM, _N_GROUPS), jnp.bfloat16),
            pltpu.VMEM((_TM, _DIM), jnp.bfloat16),
        ],
        compiler_params=pltpu.CompilerParams(
            dimension_semantics=("arbitrary",)),
    )(x, gfn)

    counts_col = counts.reshape(_N_GROUPS, 1)
    rows = 1024
    out = pl.pallas_call(
        _blend_body,
        grid=(_N_GROUPS // rows,),
        in_specs=[
            pl.BlockSpec((rows, _DIM), lambda i: (i, 0)),
            pl.BlockSpec((rows, _DIM), lambda i: (i, 0)),
            pl.BlockSpec((rows, 1), lambda i: (i, 0)),
        ],
        out_specs=pl.BlockSpec((rows, _DIM), lambda i: (i, 0)),
        out_shape=jax.ShapeDtypeStruct((_N_GROUPS, _DIM), jnp.float32),
    )(group_features, sums, counts_col)
    return out


# speedup vs baseline: 1.4526x; 1.4526x over previous
import jax
import jax.numpy as jnp
from jax.experimental import pallas as pl
from jax.experimental.pallas import tpu as pltpu

_N_GROUPS = 8192
_DIM = 256
_BETA = 0.99
_TOKENS = 65536
_TS = 256
_NSUB = 2
_TM = _TS * _NSUB


def _gfn_body(gf_ref, gfn_ref):
    gf = gf_ref[...]
    ns = jnp.sum(gf * gf, axis=1, keepdims=True)
    rnorm = 1.0 / jnp.maximum(jnp.sqrt(ns), 1e-12)
    gfn_ref[...] = (gf * rnorm).astype(jnp.bfloat16)


def _assign_accum_body(x_ref, gfn_ref, sums_ref, counts_ref):
    i = pl.program_id(0)

    @pl.when(i == 0)
    def _init():
        sums_ref[...] = jnp.zeros_like(sums_ref)
        counts_ref[...] = jnp.zeros_like(counts_ref)

    gfn = gfn_ref[...]
    dsum = None
    dcnt = None
    for k in range(_NSUB):
        xk = x_ref[k * _TS:(k + 1) * _TS, :].astype(jnp.bfloat16)
        logits = jax.lax.dot_general(
            xk, gfn, (((1,), (1,)), ((), ())),
            preferred_element_type=jnp.float32)
        rowmax = jnp.max(logits, axis=1, keepdims=True)
        oh = (logits == rowmax).astype(jnp.bfloat16)
        d = jax.lax.dot_general(
            oh, xk, (((0,), (0,)), ((), ())),
            preferred_element_type=jnp.float32)
        c = jnp.sum(oh.astype(jnp.float32), axis=0, keepdims=True)
        dsum = d if dsum is None else dsum + d
        dcnt = c if dcnt is None else dcnt + c
    sums_ref[...] += dsum
    counts_ref[...] += dcnt


def _blend_body(gf_ref, sums_ref, cnt_ref, out_ref):
    r = 1.0 / jnp.maximum(cnt_ref[...], 1.0)
    out_ref[...] = _BETA * gf_ref[...] + (1.0 - _BETA) * sums_ref[...] * r


@jax.jit
def kernel(x, group_features):
    gfn = pl.pallas_call(
        _gfn_body,
        in_specs=[pl.BlockSpec((_N_GROUPS, _DIM), lambda: (0, 0))],
        out_specs=pl.BlockSpec((_N_GROUPS, _DIM), lambda: (0, 0)),
        out_shape=jax.ShapeDtypeStruct((_N_GROUPS, _DIM), jnp.bfloat16),
    )(group_features)

    grid = _TOKENS // _TM
    sums, counts = pl.pallas_call(
        _assign_accum_body,
        grid=(grid,),
        in_specs=[
            pl.BlockSpec((_TM, _DIM), lambda i: (i, 0)),
            pl.BlockSpec((_N_GROUPS, _DIM), lambda i: (0, 0)),
        ],
        out_specs=[
            pl.BlockSpec((_N_GROUPS, _DIM), lambda i: (0, 0)),
            pl.BlockSpec((1, _N_GROUPS), lambda i: (0, 0)),
        ],
        out_shape=[
            jax.ShapeDtypeStruct((_N_GROUPS, _DIM), jnp.float32),
            jax.ShapeDtypeStruct((1, _N_GROUPS), jnp.float32),
        ],
        compiler_params=pltpu.CompilerParams(
            dimension_semantics=("arbitrary",)),
    )(x, gfn)

    counts_col = counts.reshape(_N_GROUPS, 1)
    rows = 1024
    out = pl.pallas_call(
        _blend_body,
        grid=(_N_GROUPS // rows,),
        in_specs=[
            pl.BlockSpec((rows, _DIM), lambda i: (i, 0)),
            pl.BlockSpec((rows, _DIM), lambda i: (i, 0)),
            pl.BlockSpec((rows, 1), lambda i: (i, 0)),
        ],
        out_specs=pl.BlockSpec((rows, _DIM), lambda i: (i, 0)),
        out_shape=jax.ShapeDtypeStruct((_N_GROUPS, _DIM), jnp.float32),
    )(group_features, sums, counts_col)
    return out
